# baseline (speedup 1.0000x reference)
import functools

import jax
import jax.numpy as jnp
from jax import lax
from jax.experimental import pallas as pl
from jax.experimental.pallas import tpu as pltpu
from jax.experimental.pallas import tpu_sc as plsc


@functools.lru_cache(maxsize=None)
def _make_gather(V, D, BATCH, HIST):
    info = plsc.get_sparse_core_info()
    NC, NS, L = info.num_cores, info.num_subcores, info.num_lanes
    NW = NC * NS
    assert D == 64 and L == 16
    BB = BATCH // 128
    assert BB == NW
    HT = HIST // 8
    assert HIST % 8 == 0 and HIST % 2 == 0

    mesh = plsc.VectorSubcoreMesh(core_axis_name="c", subcore_axis_name="s")

    @functools.partial(
        pl.kernel,
        mesh=mesh,
        compiler_params=pltpu.CompilerParams(
            use_tc_tiling_on_sc=False, needs_layout_passes=False),
        out_type=jax.ShapeDtypeStruct((HIST, D // 8, BB, 8, 128), jnp.float32),
        scratch_types=[
            pltpu.VMEM((HT, 8, 128), jnp.int32),
            [pltpu.VMEM((128, D), jnp.float32) for _ in range(2)],
            [pltpu.VMEM((D // 8, 8, 128), jnp.float32) for _ in range(2)],
            [pltpu.SemaphoreType.DMA for _ in range(2)],
            [pltpu.SemaphoreType.DMA for _ in range(2)],
        ],
    )
    def k(xv_hbm, table_hbm, out_hbm, idx_v, rows, tbuf, gsem, osem):
        wid = lax.axis_index("s") * NC + lax.axis_index("c")

        pltpu.sync_copy(xv_hbm.at[:, wid], idx_v)

        def gather(h, slot):
            pltpu.async_copy(
                table_hbm.at[idx_v.at[h // 8, h % 8]], rows[slot], gsem[slot])

        def wait_gather(slot):
            pltpu.make_async_copy(
                table_hbm.at[idx_v.at[0, 0]], rows[slot], gsem[slot]).wait()

        def writeout(h, slot):
            pltpu.async_copy(tbuf[slot], out_hbm.at[h, :, wid], osem[slot])

        def wait_writeout(slot):
            pltpu.make_async_copy(
                tbuf[slot], out_hbm.at[0, :, wid], osem[slot]).wait()

        def transpose(slot):
            lanes = lax.iota(jnp.int32, 16)

            def dt_body(dt, carry):
                for ds in range(8):
                    col = dt * 8 + ds
                    for blk in range(8):
                        v = plsc.load_gather(
                            rows[slot], [blk * 16 + lanes,
                                         jnp.full((16,), col, jnp.int32)])
                        tbuf[slot][dt, ds, pl.ds(blk * 16, 16)] = v
                return carry

            lax.fori_loop(0, D // 8, dt_body, 0)

        gather(0, 0)

        def pair(hh, carry):
            for par in range(2):
                h = hh * 2 + par
                slot = par

                wait_gather(slot)

                @pl.when(h + 1 < HIST)
                def _():
                    gather(h + 1, 1 - slot)

                @pl.when(h >= 2)
                def _():
                    wait_writeout(slot)

                transpose(slot)
                writeout(h, slot)
            return carry

        lax.fori_loop(0, HIST // 2, pair, 0)

        for slot in range(2):
            wait_writeout(slot)

    return k


def kernel(x, table):
    BATCH, HIST = x.shape
    V, D = table.shape
    xv = x.T.reshape(HIST // 8, 8, BATCH // 128, 128).transpose(0, 2, 1, 3)
    out5 = _make_gather(V, D, BATCH, HIST)(xv, table)
    return out5.transpose(2, 4, 0, 1, 3).reshape(BATCH, HIST, D)


# speedup vs baseline: 1.4620x; 1.4620x over previous
import functools

import jax
import jax.numpy as jnp
from jax import lax
from jax.experimental import pallas as pl
from jax.experimental.pallas import tpu as pltpu
from jax.experimental.pallas import tpu_sc as plsc


@functools.lru_cache(maxsize=None)
def _make_gather(V, D, BATCH, HIST):
    info = plsc.get_sparse_core_info()
    NC, NS, L = info.num_cores, info.num_subcores, info.num_lanes
    NW = NC * NS
    assert D == 64 and L == 16
    BB = BATCH // 128
    assert BB == NW
    HT = HIST // 8
    assert HIST % 8 == 0 and HIST % 2 == 0

    mesh = plsc.VectorSubcoreMesh(core_axis_name="c", subcore_axis_name="s")

    @functools.partial(
        pl.kernel,
        mesh=mesh,
        compiler_params=pltpu.CompilerParams(
            use_tc_tiling_on_sc=False, needs_layout_passes=False),
        out_type=jax.ShapeDtypeStruct((HIST, D // 8, BB, 8, 128), jnp.float32),
        scratch_types=[
            pltpu.VMEM((HT, 8, 128), jnp.int32),
            [pltpu.VMEM((128, D), jnp.float32) for _ in range(2)],
            [pltpu.VMEM((D // 8, 8, 128), jnp.float32) for _ in range(2)],
            [pltpu.SemaphoreType.DMA for _ in range(2)],
            [pltpu.SemaphoreType.DMA for _ in range(2)],
        ],
    )
    def k(xv_hbm, table_hbm, out_hbm, idx_v, rows, tbuf, gsem, osem):
        wid = lax.axis_index("s") * NC + lax.axis_index("c")

        pltpu.sync_copy(xv_hbm.at[:, wid], idx_v)

        def gather(h, slot):
            pltpu.async_copy(
                table_hbm.at[idx_v.at[h // 8, h % 8]], rows[slot], gsem[slot])

        def wait_gather(slot):
            pltpu.make_async_copy(
                table_hbm.at[idx_v.at[0, 0]], rows[slot], gsem[slot]).wait()

        def writeout(h, slot):
            pltpu.async_copy(tbuf[slot], out_hbm.at[h, :, wid], osem[slot])

        def wait_writeout(slot):
            pltpu.make_async_copy(
                tbuf[slot], out_hbm.at[0, :, wid], osem[slot]).wait()

        lanes = lax.iota(jnp.int32, 16)
        row_idx = [blk * 16 + lanes for blk in range(8)]

        def transpose(slot):
            @plsc.parallel_loop(0, D, unroll=4)
            def _(d):
                col = jnp.full((16,), d, jnp.int32)
                for blk in range(8):
                    v = plsc.load_gather(rows[slot], [row_idx[blk], col])
                    tbuf[slot][d // 8, d % 8, pl.ds(blk * 16, 16)] = v

        gather(0, 0)

        def pair(hh, carry):
            for par in range(2):
                h = hh * 2 + par
                slot = par

                wait_gather(slot)

                @pl.when(h + 1 < HIST)
                def _():
                    gather(h + 1, 1 - slot)

                @pl.when(h >= 2)
                def _():
                    wait_writeout(slot)

                transpose(slot)
                writeout(h, slot)
            return carry

        lax.fori_loop(0, HIST // 2, pair, 0)

        for slot in range(2):
            wait_writeout(slot)

    return k


def kernel(x, table):
    BATCH, HIST = x.shape
    V, D = table.shape
    xv = x.T.reshape(HIST // 8, 8, BATCH // 128, 128).transpose(0, 2, 1, 3)
    out5 = _make_gather(V, D, BATCH, HIST)(xv, table)
    return out5.transpose(2, 4, 0, 1, 3).reshape(BATCH, HIST, D)


# speedup vs baseline: 1.4805x; 1.0126x over previous
import functools

import jax
import jax.numpy as jnp
from jax import lax
from jax.experimental import pallas as pl
from jax.experimental.pallas import tpu as pltpu
from jax.experimental.pallas import tpu_sc as plsc

_CHUNK = 128
_NBUF = 8


@functools.lru_cache(maxsize=None)
def _make_gather(V, D, N):
    info = plsc.get_sparse_core_info()
    NC, NS = info.num_cores, info.num_subcores
    NW = NC * NS
    assert N % (NW * _CHUNK) == 0
    per_w = N // NW
    nchunks = per_w // _CHUNK

    mesh = plsc.VectorSubcoreMesh(core_axis_name="c", subcore_axis_name="s")

    @functools.partial(
        pl.kernel,
        mesh=mesh,
        compiler_params=pltpu.CompilerParams(use_tc_tiling_on_sc=False),
        out_type=jax.ShapeDtypeStruct((N, D), jnp.float32),
        scratch_types=[
            pltpu.VMEM((nchunks, _CHUNK), jnp.int32),
            [pltpu.VMEM((_CHUNK, D), jnp.float32) for _ in range(_NBUF)],
            [pltpu.SemaphoreType.DMA for _ in range(_NBUF)],
            [pltpu.SemaphoreType.DMA for _ in range(_NBUF)],
        ],
    )
    def k(xv_hbm, table_hbm, out_hbm, idx_v, rows, gsem, osem):
        wid = lax.axis_index("s") * NC + lax.axis_index("c")
        base = wid * per_w

        pltpu.sync_copy(xv_hbm.at[wid], idx_v)

        def gather(c, slot):
            pltpu.async_copy(
                table_hbm.at[idx_v.at[c]], rows[slot], gsem[slot])

        def wait_gather(slot):
            pltpu.make_async_copy(
                table_hbm.at[idx_v.at[0]], rows[slot], gsem[slot]).wait()

        def writeout(c, slot):
            pltpu.async_copy(
                rows[slot], out_hbm.at[pl.ds(base + c * _CHUNK, _CHUNK)],
                osem[slot])

        def wait_writeout(slot):
            pltpu.make_async_copy(
                rows[slot], out_hbm.at[pl.ds(0, _CHUNK)], osem[slot]).wait()

        for i in range(_NBUF):
            gather(i, i)

        def group(grp, carry):
            for par in range(_NBUF):
                g = grp * _NBUF + par
                wait_gather(par)
                writeout(g, par)

                @pl.when(g + _NBUF < nchunks)
                def _():
                    wait_writeout(par)
                    gather(g + _NBUF, par)

            return carry

        assert nchunks % _NBUF == 0
        lax.fori_loop(0, nchunks // _NBUF, group, 0)

        for slot in range(_NBUF):
            wait_writeout(slot)

    return k


def kernel(x, table):
    BATCH, HIST = x.shape
    V, D = table.shape
    N = BATCH * HIST
    info = plsc.get_sparse_core_info()
    NW = info.num_cores * info.num_subcores
    xv = x.reshape(NW, N // (NW * _CHUNK), _CHUNK)
    out = _make_gather(V, D, N)(xv, table)
    return out.reshape(BATCH, HIST, D)
